# baseline (speedup 1.0000x reference)
import functools

import jax
import jax.numpy as jnp
from jax.experimental import pallas as pl
from jax.experimental.pallas import tpu as pltpu

_VB = 32768


def _pass_a(x_ref, bid_out, max_out, vmax_ref, vbid_ref, *, B, V, NB):
    i = pl.program_id(0)

    @pl.when(i == 0)
    def _init():
        vmax_ref[...] = jnp.full((B, 1), -jnp.inf, jnp.float32)
        vbid_ref[...] = jnp.zeros((B, 1), jnp.int32)

    def _update(bmax):
        better = bmax > vmax_ref[...]
        vbid_ref[...] = jnp.where(better, i, vbid_ref[...])
        vmax_ref[...] = jnp.where(better, bmax, vmax_ref[...])

    @pl.when(i < NB - 1)
    def _full():
        _update(jnp.max(x_ref[...].reshape(B, _VB), axis=1, keepdims=True))

    @pl.when(i == NB - 1)
    def _tail():
        rem = V - (NB - 1) * _VB
        lidx = jax.lax.broadcasted_iota(jnp.int32, (B, _VB), 1)
        x = jnp.where(lidx < rem, x_ref[...].reshape(B, _VB), -jnp.inf)
        _update(jnp.max(x, axis=1, keepdims=True))
        bid_out[...] = vbid_ref[...]
        max_out[...] = vmax_ref[...]


def _pass_b(bid_sref, gi_ref, x_any, bidv_ref, max_ref, lti_ref, am_ref, gt_ref,
            tok_out, lti_out, am_out, gt_out, gi_out, xbuf, tbuf, sem, *, B, V, S):
    amax = ((V - _VB) // 128) * 128
    tw = 128 + (V % 128 or 128)
    toff = V - tw
    copies = []
    for b in range(B):
        off = pl.multiple_of(jnp.minimum(bid_sref[b] * _VB, amax), 128)
        copies.append(pltpu.make_async_copy(
            x_any.at[b, 0, pl.ds(off, _VB)], xbuf.at[b], sem))
        copies.append(pltpu.make_async_copy(
            x_any.at[b, 0, pl.ds(toff, tw)], tbuf.at[b], sem))
    for c in copies:
        c.start()
    for c in copies:
        c.wait()

    big = jnp.int32(2**31 - 1)
    x = xbuf[...]
    base = jnp.minimum(bidv_ref[...] * _VB, amax)
    lidx = jax.lax.broadcasted_iota(jnp.int32, (B, _VB), 1)
    cand = jnp.where(x == max_ref[...], lidx + base, big)
    m1 = jnp.min(cand, axis=1, keepdims=True)
    t = tbuf[...]
    tidx = jax.lax.broadcasted_iota(jnp.int32, (B, tw), 1) + toff
    cand2 = jnp.where(t == max_ref[...], tidx, big)
    m2 = jnp.min(cand2, axis=1, keepdims=True)
    tokens = jnp.minimum(m1, m2)
    tok_out[...] = tokens
    lti = jnp.minimum(lti_ref[...] + 1, S - 1)
    lti_out[...] = lti
    scol = jax.lax.broadcasted_iota(jnp.int32, (B, S), 1)
    am_out[...] = jnp.where(scol == lti, 1, am_ref[...])
    gi = gi_ref[0]
    gt_out[...] = jnp.where(scol == gi, tokens, gt_ref[...])
    gi_out[0] = jnp.minimum(gi + 1, S - 1)


def kernel(logits, last_token_index, attention_mask, generated_tokens, generated_index):
    B, _, V = logits.shape
    S = generated_tokens.shape[1]
    NB = pl.cdiv(V, _VB)

    bid, vmax = pl.pallas_call(
        functools.partial(_pass_a, B=B, V=V, NB=NB),
        grid=(NB,),
        in_specs=[pl.BlockSpec((B, 1, _VB), lambda i: (0, 0, i))],
        out_specs=[
            pl.BlockSpec((B, 1), lambda i: (0, 0)),
            pl.BlockSpec((B, 1), lambda i: (0, 0)),
        ],
        out_shape=(
            jax.ShapeDtypeStruct((B, 1), jnp.int32),
            jax.ShapeDtypeStruct((B, 1), jnp.float32),
        ),
        scratch_shapes=[
            pltpu.VMEM((B, 1), jnp.float32),
            pltpu.VMEM((B, 1), jnp.int32),
        ],
        compiler_params=pltpu.CompilerParams(
            dimension_semantics=("arbitrary",),
        ),
    )(logits)

    const = lambda i, bid_ref, gi_ref: (0, 0)
    grid_spec = pltpu.PrefetchScalarGridSpec(
        num_scalar_prefetch=2,
        grid=(1,),
        in_specs=[
            pl.BlockSpec(memory_space=pl.ANY),
            pl.BlockSpec((B, 1), const),
            pl.BlockSpec((B, 1), const),
            pl.BlockSpec((B, 1), const),
            pl.BlockSpec((B, S), const),
            pl.BlockSpec((B, S), const),
        ],
        out_specs=[
            pl.BlockSpec((B, 1), const),
            pl.BlockSpec((B, 1), const),
            pl.BlockSpec((B, S), const),
            pl.BlockSpec((B, S), const),
            pl.BlockSpec(memory_space=pltpu.SMEM),
        ],
        scratch_shapes=[
            pltpu.VMEM((B, _VB), jnp.float32),
            pltpu.VMEM((B, 128 + (V % 128 or 128)), jnp.float32),
            pltpu.SemaphoreType.DMA,
        ],
    )
    tok, lti, am, gt, gi = pl.pallas_call(
        functools.partial(_pass_b, B=B, V=V, S=S),
        grid_spec=grid_spec,
        out_shape=(
            jax.ShapeDtypeStruct((B, 1), jnp.int32),
            jax.ShapeDtypeStruct((B, 1), jnp.int32),
            jax.ShapeDtypeStruct((B, S), attention_mask.dtype),
            jax.ShapeDtypeStruct((B, S), generated_tokens.dtype),
            jax.ShapeDtypeStruct((1,), jnp.int32),
        ),
        compiler_params=pltpu.CompilerParams(
            dimension_semantics=("arbitrary",),
        ),
    )(bid.reshape(B), generated_index, logits, bid, vmax, last_token_index,
      attention_mask, generated_tokens)
    return tok, lti, am, gt, gi


# speedup vs baseline: 1.0934x; 1.0934x over previous
import functools

import jax
import jax.numpy as jnp
from jax.experimental import pallas as pl
from jax.experimental.pallas import tpu as pltpu

_VB = 65536


def _pass_a(x_ref, bid_out, max_out, vmax_ref, vbid_ref, *, B, V, NB):
    i = pl.program_id(0)

    @pl.when(i == 0)
    def _init():
        vmax_ref[...] = jnp.full((B, 1), -jnp.inf, jnp.float32)
        vbid_ref[...] = jnp.zeros((B, 1), jnp.int32)

    def _update(bmax):
        better = bmax > vmax_ref[...]
        vbid_ref[...] = jnp.where(better, i, vbid_ref[...])
        vmax_ref[...] = jnp.where(better, bmax, vmax_ref[...])

    @pl.when(i < NB - 1)
    def _full():
        _update(jnp.max(x_ref[...].reshape(B, _VB), axis=1, keepdims=True))

    @pl.when(i == NB - 1)
    def _tail():
        rem = V - (NB - 1) * _VB
        lidx = jax.lax.broadcasted_iota(jnp.int32, (B, _VB), 1)
        x = jnp.where(lidx < rem, x_ref[...].reshape(B, _VB), -jnp.inf)
        _update(jnp.max(x, axis=1, keepdims=True))
        bid_out[...] = vbid_ref[...]
        max_out[...] = vmax_ref[...]


def _pass_b(bid_sref, gi_ref, x_any, bidv_ref, max_ref, lti_ref, am_ref, gt_ref,
            tok_out, lti_out, am_out, gt_out, gi_out, xbuf, tbuf, sem, *, B, V, S):
    amax = ((V - _VB) // 128) * 128
    tw = 128 + (V % 128 or 128)
    toff = V - tw
    copies = []
    for b in range(B):
        off = pl.multiple_of(jnp.minimum(bid_sref[b] * _VB, amax), 128)
        copies.append(pltpu.make_async_copy(
            x_any.at[b, 0, pl.ds(off, _VB)], xbuf.at[b], sem))
        copies.append(pltpu.make_async_copy(
            x_any.at[b, 0, pl.ds(toff, tw)], tbuf.at[b], sem))
    for c in copies:
        c.start()
    for c in copies:
        c.wait()

    big = jnp.int32(2**31 - 1)
    x = xbuf[...]
    base = jnp.minimum(bidv_ref[...] * _VB, amax)
    lidx = jax.lax.broadcasted_iota(jnp.int32, (B, _VB), 1)
    cand = jnp.where(x == max_ref[...], lidx + base, big)
    m1 = jnp.min(cand, axis=1, keepdims=True)
    t = tbuf[...]
    tidx = jax.lax.broadcasted_iota(jnp.int32, (B, tw), 1) + toff
    cand2 = jnp.where(t == max_ref[...], tidx, big)
    m2 = jnp.min(cand2, axis=1, keepdims=True)
    tokens = jnp.minimum(m1, m2)
    tok_out[...] = tokens
    lti = jnp.minimum(lti_ref[...] + 1, S - 1)
    lti_out[...] = lti
    scol = jax.lax.broadcasted_iota(jnp.int32, (B, S), 1)
    am_out[...] = jnp.where(scol == lti, 1, am_ref[...])
    gi = gi_ref[0]
    gt_out[...] = jnp.where(scol == gi, tokens, gt_ref[...])
    gi_out[0] = jnp.minimum(gi + 1, S - 1)


def kernel(logits, last_token_index, attention_mask, generated_tokens, generated_index):
    B, _, V = logits.shape
    S = generated_tokens.shape[1]
    NB = pl.cdiv(V, _VB)

    bid, vmax = pl.pallas_call(
        functools.partial(_pass_a, B=B, V=V, NB=NB),
        grid=(NB,),
        in_specs=[pl.BlockSpec((B, 1, _VB), lambda i: (0, 0, i))],
        out_specs=[
            pl.BlockSpec((B, 1), lambda i: (0, 0)),
            pl.BlockSpec((B, 1), lambda i: (0, 0)),
        ],
        out_shape=(
            jax.ShapeDtypeStruct((B, 1), jnp.int32),
            jax.ShapeDtypeStruct((B, 1), jnp.float32),
        ),
        scratch_shapes=[
            pltpu.VMEM((B, 1), jnp.float32),
            pltpu.VMEM((B, 1), jnp.int32),
        ],
        compiler_params=pltpu.CompilerParams(
            dimension_semantics=("arbitrary",),
        ),
    )(logits)

    const = lambda i, bid_ref, gi_ref: (0, 0)
    grid_spec = pltpu.PrefetchScalarGridSpec(
        num_scalar_prefetch=2,
        grid=(1,),
        in_specs=[
            pl.BlockSpec(memory_space=pl.ANY),
            pl.BlockSpec((B, 1), const),
            pl.BlockSpec((B, 1), const),
            pl.BlockSpec((B, 1), const),
            pl.BlockSpec((B, S), const),
            pl.BlockSpec((B, S), const),
        ],
        out_specs=[
            pl.BlockSpec((B, 1), const),
            pl.BlockSpec((B, 1), const),
            pl.BlockSpec((B, S), const),
            pl.BlockSpec((B, S), const),
            pl.BlockSpec(memory_space=pltpu.SMEM),
        ],
        scratch_shapes=[
            pltpu.VMEM((B, _VB), jnp.float32),
            pltpu.VMEM((B, 128 + (V % 128 or 128)), jnp.float32),
            pltpu.SemaphoreType.DMA,
        ],
    )
    tok, lti, am, gt, gi = pl.pallas_call(
        functools.partial(_pass_b, B=B, V=V, S=S),
        grid_spec=grid_spec,
        out_shape=(
            jax.ShapeDtypeStruct((B, 1), jnp.int32),
            jax.ShapeDtypeStruct((B, 1), jnp.int32),
            jax.ShapeDtypeStruct((B, S), attention_mask.dtype),
            jax.ShapeDtypeStruct((B, S), generated_tokens.dtype),
            jax.ShapeDtypeStruct((1,), jnp.int32),
        ),
        compiler_params=pltpu.CompilerParams(
            dimension_semantics=("arbitrary",),
        ),
    )(bid.reshape(B), generated_index, logits, bid, vmax, last_token_index,
      attention_mask, generated_tokens)
    return tok, lti, am, gt, gi
